# baseline (speedup 1.0000x reference)
import functools

import jax
import jax.numpy as jnp
from jax.experimental import pallas as pl
from jax.experimental.pallas import tpu as pltpu

N = 8192
HIDDEN = 1024
NUM_SPACES = 16
TOP_K = 2
BM = 512
NBLK = N // BM
NEG = -1000000000.0

_HIGH = jax.lax.Precision.HIGHEST


def _gelu(x):
    return 0.5 * x * (1.0 + jax.lax.erf(x * 0.7071067811865476))


def _dot(a, b):
    return jax.lax.dot_general(a, b, (((1,), (0,)), ((), ())),
                               precision=_HIGH, preferred_element_type=jnp.float32)


def _router_kernel(sm_ref, tm_ref, ds_ref, w1_ref, b1_ref, w2_ref, b2_ref,
                   wd1_ref, bd1_ref, wd2_ref, bd2_ref,
                   probs_ref, beta_ref, sval_ref, sfnd_ref):
    t = pl.program_id(0)

    @pl.when(t == 0)
    def _init():
        sval_ref[...] = jnp.full((NUM_SPACES, NUM_SPACES), NEG, jnp.float32)
        sfnd_ref[...] = jnp.zeros((NUM_SPACES, NUM_SPACES), jnp.float32)

    sm = sm_ref[...]
    tm = tm_ref[...]
    ds = ds_ref[...]

    h = (_dot(sm, w1_ref[0:HIDDEN, :])
         + _dot(tm, w1_ref[HIDDEN:2 * HIDDEN, :])
         + _dot(ds, w1_ref[2 * HIDDEN:3 * HIDDEN, :])
         + b1_ref[...])
    h = _gelu(h)
    logits = _dot(h, w2_ref[...]) + b2_ref[...]

    hd = (_dot(sm, wd1_ref[0:HIDDEN, :])
          + _dot(tm, wd1_ref[HIDDEN:2 * HIDDEN, :])
          + _dot(ds, wd1_ref[2 * HIDDEN:3 * HIDDEN, :])
          + bd1_ref[...])
    hd = _gelu(hd)
    beta_ref[...] = jax.nn.sigmoid(_dot(hd, wd2_ref[...]) + bd2_ref[...])

    lane = jax.lax.broadcasted_iota(jnp.int32, (BM, NUM_SPACES), 1)
    m1 = jnp.max(logits, axis=1, keepdims=True)
    i1 = jnp.min(jnp.where(logits == m1, lane, NUM_SPACES), axis=1, keepdims=True)
    rest = jnp.where(lane == i1, -3.0e38, logits)
    m2 = jnp.max(rest, axis=1, keepdims=True)
    i2 = jnp.min(jnp.where(rest == m2, lane, NUM_SPACES), axis=1, keepdims=True)

    rowid = jax.lax.broadcasted_iota(jnp.int32, (BM, NUM_SPACES), 0)
    ones = jnp.ones((BM, 1), jnp.float32)
    for j, (ij, vj) in enumerate(((i1, m1), (i2, m2))):
        onehot = ij == lane
        best_row = jnp.max(jnp.where(onehot, rowid, -1), axis=0, keepdims=True)
        sel = jnp.where(onehot & (rowid == best_row), 1.0, 0.0)
        r = jax.lax.dot_general(sel, jnp.concatenate([vj, ones], axis=1),
                                (((0,), (0,)), ((), ())),
                                precision=_HIGH, preferred_element_type=jnp.float32)
        cur_f = sfnd_ref[:, j:j + 1]
        sval_ref[:, j:j + 1] = jnp.where(cur_f > 0, sval_ref[:, j:j + 1], r[:, 0:1])
        sfnd_ref[:, j:j + 1] = cur_f + r[:, 1:2]

    @pl.when(t != NBLK - 1)
    def _uniform():
        probs_ref[...] = jnp.full((BM, NUM_SPACES), 1.0 / NUM_SPACES, jnp.float32)

    @pl.when(t == NBLK - 1)
    def _finalize():
        sparse = jnp.where(sfnd_ref[...] > 0, sval_ref[...], NEG)
        mx = jnp.max(sparse, axis=1, keepdims=True)
        e = jnp.exp(sparse - mx)
        p16 = e / jnp.sum(e, axis=1, keepdims=True)
        probs_ref[0:NUM_SPACES, :] = p16
        probs_ref[NUM_SPACES:BM, :] = jnp.full(
            (BM - NUM_SPACES, NUM_SPACES), 1.0 / NUM_SPACES, jnp.float32)


@functools.partial(jax.jit, static_argnames=())
def kernel(static_mean, temporal_mean, disagreement, W1, b1, W2, b2, Wd1, bd1, Wd2, bd2):
    b1r = b1.reshape(1, HIDDEN)
    b2r = b2.reshape(1, NUM_SPACES)
    bd1r = bd1.reshape(1, HIDDEN // 2)
    bd2r = bd2.reshape(1, 1)

    rev = lambda t: (NBLK - 1 - t, 0)
    fixed = lambda t: (0, 0)
    probs, beta = pl.pallas_call(
        _router_kernel,
        grid=(NBLK,),
        in_specs=[
            pl.BlockSpec((BM, HIDDEN), rev),
            pl.BlockSpec((BM, HIDDEN), rev),
            pl.BlockSpec((BM, HIDDEN), rev),
            pl.BlockSpec((3 * HIDDEN, HIDDEN), fixed),
            pl.BlockSpec((1, HIDDEN), fixed),
            pl.BlockSpec((HIDDEN, NUM_SPACES), fixed),
            pl.BlockSpec((1, NUM_SPACES), fixed),
            pl.BlockSpec((3 * HIDDEN, HIDDEN // 2), fixed),
            pl.BlockSpec((1, HIDDEN // 2), fixed),
            pl.BlockSpec((HIDDEN // 2, 1), fixed),
            pl.BlockSpec((1, 1), fixed),
        ],
        out_specs=[
            pl.BlockSpec((BM, NUM_SPACES), rev),
            pl.BlockSpec((BM, 1), rev),
        ],
        out_shape=[
            jax.ShapeDtypeStruct((N, NUM_SPACES), jnp.float32),
            jax.ShapeDtypeStruct((N, 1), jnp.float32),
        ],
        scratch_shapes=[
            pltpu.VMEM((NUM_SPACES, NUM_SPACES), jnp.float32),
            pltpu.VMEM((NUM_SPACES, NUM_SPACES), jnp.float32),
        ],
        compiler_params=pltpu.CompilerParams(
            dimension_semantics=("arbitrary",),
        ),
    )(static_mean, temporal_mean, disagreement, W1, b1r, W2, b2r,
      Wd1, bd1r, Wd2, bd2r)
    return probs, beta[:, 0]


# speedup vs baseline: 2.2811x; 2.2811x over previous
import functools

import jax
import jax.numpy as jnp
from jax.experimental import pallas as pl
from jax.experimental.pallas import tpu as pltpu

N = 8192
HIDDEN = 1024
NUM_SPACES = 16
TOP_K = 2
BM = 512
NBLK = N // BM
NEG = -1000000000.0

_HIGH = jax.lax.Precision.HIGHEST


def _gelu(x):
    return 0.5 * x * (1.0 + jax.lax.erf(x * 0.7071067811865476))


def _dotb(a, b):
    return jax.lax.dot_general(a, b, (((1,), (0,)), ((), ())),
                               preferred_element_type=jnp.float32)


def _split(x):
    hi = x.astype(jnp.bfloat16)
    lo = (x - hi.astype(jnp.float32)).astype(jnp.bfloat16)
    return hi, lo


def _dot3(xhi, xlo, whi, wlo):
    return _dotb(xhi, whi) + _dotb(xhi, wlo) + _dotb(xlo, whi)


def _router_kernel(sm_ref, tm_ref, ds_ref, w1hi_ref, w1lo_ref, b1_ref,
                   w2hi_ref, w2lo_ref, b2_ref,
                   wd1_ref, bd1_ref, wd2_ref, bd2_ref,
                   probs_ref, beta_ref, sval_ref, sfnd_ref):
    t = pl.program_id(0)

    @pl.when(t == 0)
    def _init():
        sval_ref[...] = jnp.full((NUM_SPACES, NUM_SPACES), NEG, jnp.float32)
        sfnd_ref[...] = jnp.zeros((NUM_SPACES, NUM_SPACES), jnp.float32)

    smhi, smlo = _split(sm_ref[...])
    tmhi, tmlo = _split(tm_ref[...])
    dshi, dslo = _split(ds_ref[...])
    xhi = jnp.concatenate([smhi, tmhi, dshi], axis=1)
    xlo = jnp.concatenate([smlo, tmlo, dslo], axis=1)

    h = _dot3(xhi, xlo, w1hi_ref[...], w1lo_ref[...]) + b1_ref[...]
    h = _gelu(h)
    hhi, hlo = _split(h)
    logits = _dot3(hhi, hlo, w2hi_ref[...], w2lo_ref[...]) + b2_ref[...]

    hd = _gelu(_dotb(xhi, wd1_ref[...]) + bd1_ref[...])
    beta_ref[...] = jax.nn.sigmoid(
        _dotb(hd.astype(jnp.bfloat16), wd2_ref[...]) + bd2_ref[...])

    lane = jax.lax.broadcasted_iota(jnp.int32, (BM, NUM_SPACES), 1)
    m1 = jnp.max(logits, axis=1, keepdims=True)
    i1 = jnp.min(jnp.where(logits == m1, lane, NUM_SPACES), axis=1, keepdims=True)
    rest = jnp.where(lane == i1, -3.0e38, logits)
    m2 = jnp.max(rest, axis=1, keepdims=True)
    i2 = jnp.min(jnp.where(rest == m2, lane, NUM_SPACES), axis=1, keepdims=True)

    rowid = jax.lax.broadcasted_iota(jnp.int32, (BM, NUM_SPACES), 0)
    ones = jnp.ones((BM, 1), jnp.float32)
    for j, (ij, vj) in enumerate(((i1, m1), (i2, m2))):
        onehot = ij == lane
        best_row = jnp.max(jnp.where(onehot, rowid, -1), axis=0, keepdims=True)
        sel = jnp.where(onehot & (rowid == best_row), 1.0, 0.0)
        r = jax.lax.dot_general(sel, jnp.concatenate([vj, ones], axis=1),
                                (((0,), (0,)), ((), ())),
                                precision=_HIGH, preferred_element_type=jnp.float32)
        cur_f = sfnd_ref[:, j:j + 1]
        sval_ref[:, j:j + 1] = jnp.where(cur_f > 0, sval_ref[:, j:j + 1], r[:, 0:1])
        sfnd_ref[:, j:j + 1] = cur_f + r[:, 1:2]

    @pl.when(t != NBLK - 1)
    def _uniform():
        probs_ref[...] = jnp.full((BM, NUM_SPACES), 1.0 / NUM_SPACES, jnp.float32)

    @pl.when(t == NBLK - 1)
    def _finalize():
        sparse = jnp.where(sfnd_ref[...] > 0, sval_ref[...], NEG)
        mx = jnp.max(sparse, axis=1, keepdims=True)
        e = jnp.exp(sparse - mx)
        p16 = e / jnp.sum(e, axis=1, keepdims=True)
        probs_ref[0:NUM_SPACES, :] = p16
        probs_ref[NUM_SPACES:BM, :] = jnp.full(
            (BM - NUM_SPACES, NUM_SPACES), 1.0 / NUM_SPACES, jnp.float32)


@functools.partial(jax.jit, static_argnames=())
def kernel(static_mean, temporal_mean, disagreement, W1, b1, W2, b2, Wd1, bd1, Wd2, bd2):
    w1hi = W1.astype(jnp.bfloat16)
    w1lo = (W1 - w1hi.astype(jnp.float32)).astype(jnp.bfloat16)
    w2hi = W2.astype(jnp.bfloat16)
    w2lo = (W2 - w2hi.astype(jnp.float32)).astype(jnp.bfloat16)
    wd1b = Wd1.astype(jnp.bfloat16)
    wd2b = Wd2.astype(jnp.bfloat16)
    b1r = b1.reshape(1, HIDDEN)
    b2r = b2.reshape(1, NUM_SPACES)
    bd1r = bd1.reshape(1, HIDDEN // 2)
    bd2r = bd2.reshape(1, 1)

    rev = lambda t: (NBLK - 1 - t, 0)
    fixed = lambda t: (0, 0)
    probs, beta = pl.pallas_call(
        _router_kernel,
        grid=(NBLK,),
        in_specs=[
            pl.BlockSpec((BM, HIDDEN), rev),
            pl.BlockSpec((BM, HIDDEN), rev),
            pl.BlockSpec((BM, HIDDEN), rev),
            pl.BlockSpec((3 * HIDDEN, HIDDEN), fixed),
            pl.BlockSpec((3 * HIDDEN, HIDDEN), fixed),
            pl.BlockSpec((1, HIDDEN), fixed),
            pl.BlockSpec((HIDDEN, NUM_SPACES), fixed),
            pl.BlockSpec((HIDDEN, NUM_SPACES), fixed),
            pl.BlockSpec((1, NUM_SPACES), fixed),
            pl.BlockSpec((3 * HIDDEN, HIDDEN // 2), fixed),
            pl.BlockSpec((1, HIDDEN // 2), fixed),
            pl.BlockSpec((HIDDEN // 2, 1), fixed),
            pl.BlockSpec((1, 1), fixed),
        ],
        out_specs=[
            pl.BlockSpec((BM, NUM_SPACES), rev),
            pl.BlockSpec((BM, 1), rev),
        ],
        out_shape=[
            jax.ShapeDtypeStruct((N, NUM_SPACES), jnp.float32),
            jax.ShapeDtypeStruct((N, 1), jnp.float32),
        ],
        scratch_shapes=[
            pltpu.VMEM((NUM_SPACES, NUM_SPACES), jnp.float32),
            pltpu.VMEM((NUM_SPACES, NUM_SPACES), jnp.float32),
        ],
        compiler_params=pltpu.CompilerParams(
            dimension_semantics=("arbitrary",),
        ),
    )(static_mean, temporal_mean, disagreement, w1hi, w1lo, b1r,
      w2hi, w2lo, b2r, wd1b, bd1r, wd2b, bd2r)
    return probs, beta[:, 0]
